# Initial kernel scaffold; baseline (speedup 1.0000x reference)
import jax
import jax.numpy as jnp
from jax.experimental import pallas as pl


def kernel(src, pos_table):
    raise NotImplementedError("write your pallas kernel here")


# speedup vs baseline: 4.3207x; 4.3207x over previous
import jax
import jax.numpy as jnp
from jax.experimental import pallas as pl

_PAD = 0


def _copy_body(pos_ref, out_ref):
    out_ref[0] = pos_ref[...]


def _mask_body(src_ref, mask_ref):
    mask_ref[...] = (src_ref[...] == _PAD)


def kernel(src, pos_table):
    N, S = src.shape
    _, E = pos_table.shape
    BS = 2048

    pos_emb = pl.pallas_call(
        _copy_body,
        grid=(S // BS, N),
        in_specs=[pl.BlockSpec((BS, E), lambda j, i: (j, 0))],
        out_specs=pl.BlockSpec((1, BS, E), lambda j, i: (i, j, 0)),
        out_shape=jax.ShapeDtypeStruct((N, S, E), pos_table.dtype),
    )(pos_table)

    mask = pl.pallas_call(
        _mask_body,
        out_shape=jax.ShapeDtypeStruct((N, S), jnp.bool_),
    )(src)
    return pos_emb, mask
